# baseline (speedup 1.0000x reference)
import functools

import jax
import jax.numpy as jnp
from jax import lax
from jax.experimental import pallas as pl
from jax.experimental.pallas import tpu as pltpu
from jax.experimental.pallas import tpu_sc as plsc

NUM_EMBEDDINGS = 1000000
D = 64
BATCH = 4096
SEQ_LEN = 200
B = BATCH * SEQ_LEN

NC = 2
NS = 16
NW = NC * NS
BPW = B // NW

CH = 1024
GPG = 1
GROUP = CH * GPG
NGROUPS = BPW // GROUP


def _make_kernel():
    mesh = plsc.VectorSubcoreMesh(core_axis_name="c", subcore_axis_name="s")

    @functools.partial(
        pl.kernel,
        out_type=jax.ShapeDtypeStruct((B, D), jnp.float32),
        mesh=mesh,
        scratch_types=[
            pltpu.VMEM((GROUP,), jnp.int32),
            pltpu.VMEM((GROUP, D), jnp.float32),
            pltpu.SemaphoreType.DMA,
        ],
        compiler_params=pltpu.CompilerParams(use_tc_tiling_on_sc=False),
    )
    def emb_kernel(idx_hbm, table_hbm, out_hbm, idx_v, rows_v, sem):
        wid = lax.axis_index("s") * NC + lax.axis_index("c")
        base = wid * BPW

        def body(g, carry):
            row0 = base + g * GROUP
            pltpu.sync_copy(idx_hbm.at[pl.ds(row0, GROUP)], idx_v)
            copies = []
            for j in range(GPG):
                copies.append(
                    pltpu.async_copy(
                        table_hbm.at[idx_v.at[pl.ds(j * CH, CH)]],
                        rows_v.at[pl.ds(j * CH, CH)],
                        sem,
                    )
                )
            for c in copies:
                c.wait()
            pltpu.sync_copy(rows_v, out_hbm.at[pl.ds(row0, GROUP)])
            return carry

        lax.fori_loop(0, NGROUPS, body, None)

    return emb_kernel


_emb_kernel = _make_kernel()


def kernel(input_, W):
    idx = input_.reshape(B).astype(jnp.int32)
    out = _emb_kernel(idx, W)
    return out.reshape(BATCH, SEQ_LEN, D)


# speedup vs baseline: 1.0120x; 1.0120x over previous
import functools

import jax
import jax.numpy as jnp
from jax import lax
from jax.experimental import pallas as pl
from jax.experimental.pallas import tpu as pltpu
from jax.experimental.pallas import tpu_sc as plsc

NUM_EMBEDDINGS = 1000000
D = 64
BATCH = 4096
SEQ_LEN = 200
B = BATCH * SEQ_LEN

NC = 2
NS = 16
NW = NC * NS
BPW = B // NW

GROUP = 800
NG = BPW // GROUP
NPAIR = NG // 2


def _make_kernel():
    mesh = plsc.VectorSubcoreMesh(core_axis_name="c", subcore_axis_name="s")

    @functools.partial(
        pl.kernel,
        out_type=jax.ShapeDtypeStruct((B, D), jnp.float32),
        mesh=mesh,
        scratch_types=[
            pltpu.VMEM((GROUP,), jnp.int32),
            pltpu.VMEM((GROUP,), jnp.int32),
            pltpu.VMEM((GROUP, D), jnp.float32),
            pltpu.VMEM((GROUP, D), jnp.float32),
            pltpu.SemaphoreType.DMA,
            pltpu.SemaphoreType.DMA,
            pltpu.SemaphoreType.DMA,
            pltpu.SemaphoreType.DMA,
            pltpu.SemaphoreType.DMA,
            pltpu.SemaphoreType.DMA,
        ],
        compiler_params=pltpu.CompilerParams(use_tc_tiling_on_sc=False),
    )
    def emb_kernel(idx_hbm, table_hbm, out_hbm,
                   idx0, idx1, rows0, rows1,
                   isem0, isem1, gsem0, gsem1, wsem0, wsem1):
        wid = lax.axis_index("s") * NC + lax.axis_index("c")
        base = wid * BPW
        idx_v = (idx0, idx1)
        rows_v = (rows0, rows1)
        isem = (isem0, isem1)
        gsem = (gsem0, gsem1)
        wsem = (wsem0, wsem1)

        def idx_copy(g, b):
            return pltpu.make_async_copy(
                idx_hbm.at[pl.ds(base + g * GROUP, GROUP)], idx_v[b], isem[b])

        def gather(b):
            return pltpu.make_async_copy(
                table_hbm.at[idx_v[b]], rows_v[b], gsem[b])

        def writeback(g, b):
            return pltpu.make_async_copy(
                rows_v[b], out_hbm.at[pl.ds(base + g * GROUP, GROUP)], wsem[b])

        idx_copy(0, 0).start()
        idx_copy(1, 1).start()

        def pair(p, carry):
            for b in range(2):
                g = 2 * p + b

                @pl.when(p > 0)
                def _wait_prev_wb():
                    writeback(g - 2, b).wait()

                idx_copy(g, b).wait()
                gather(b).start()
                gather(b).wait()

                @pl.when(g + 2 < NG)
                def _prefetch_idx():
                    idx_copy(g + 2, b).start()

                writeback(g, b).start()
            return carry

        lax.fori_loop(0, NPAIR, pair, None)
        writeback(NG - 2, 0).wait()
        writeback(NG - 1, 1).wait()

    return emb_kernel


_emb_kernel = _make_kernel()


def kernel(input_, W):
    idx = input_.reshape(B).astype(jnp.int32)
    out = _emb_kernel(idx, W)
    return out.reshape(BATCH, SEQ_LEN, D)
